# baseline (speedup 1.0000x reference)
import functools

import jax
import jax.numpy as jnp
from jax import lax
from jax.experimental import pallas as pl
from jax.experimental.pallas import tpu as pltpu
from jax.experimental.pallas import tpu_sc as plsc

NUM_USERS = 5000
NUM_ITEMS = 5000
N = NUM_USERS + NUM_ITEMS
E = 320000
D = 128

NC = 2
NS = 16
NW = NC * NS
EPW = E // NW
C = 80
NCHUNK = EPW // C
SEG = 25
NSEG = NCHUNK // SEG
RPS = N // NS


@functools.cache
def _make_sc_aggregate():
    mesh = plsc.VectorSubcoreMesh(core_axis_name="c", subcore_axis_name="s")

    @functools.partial(
        pl.kernel,
        mesh=mesh,
        out_type=jax.ShapeDtypeStruct((NC, NS, RPS, D), jnp.float32),
        scratch_types=[
            pltpu.VMEM((SEG, C), jnp.int32),
            pltpu.VMEM((SEG, C), jnp.int32),
            pltpu.VMEM((SEG, C), jnp.float32),
            pltpu.VMEM((2, C, D), jnp.float32),
            pltpu.VMEM_SHARED((N, D), jnp.float32),
            pltpu.SemaphoreType.DMA,
        ],
    )
    def _sc_aggregate(ego_hbm, src_hbm, dst_hbm, w_hbm, out_hbm,
                      src_v, dst_v, w_v, rows_v, acc_sh, sem):
        c = lax.axis_index("c")
        s = lax.axis_index("s")
        wid = c * NS + s

        zvec = jnp.zeros((16,), jnp.float32)

        def _zrow(i, carry):
            for k in range(D // 16):
                rows_v[0, i, pl.ds(k * 16, 16)] = zvec
            return carry

        lax.fori_loop(0, C, _zrow, 0)
        base = s * RPS
        for q in range(RPS // C):
            pltpu.sync_copy(rows_v.at[0], acc_sh.at[pl.ds(base + q * C, C)])
        rem = RPS % C
        if rem:
            pltpu.sync_copy(rows_v.at[0, pl.ds(0, rem)],
                            acc_sh.at[pl.ds(base + (RPS // C) * C, rem)])
        plsc.subcore_barrier()

        for seg in range(NSEG):
            sg = wid * NSEG + seg
            cp_src = pltpu.async_copy(src_hbm.at[sg], src_v, sem)
            cp_dst = pltpu.async_copy(dst_hbm.at[sg], dst_v, sem)
            cp_w = pltpu.async_copy(w_hbm.at[sg], w_v, sem)
            cp_src.wait()
            cp_dst.wait()
            cp_w.wait()

            pltpu.async_copy(ego_hbm.at[src_v.at[0]], rows_v.at[0], sem)

            def _chunk(j, carry):
                b = lax.rem(j, 2)

                @pl.when(j + 1 < SEG)
                def _prefetch():
                    pltpu.async_copy(
                        ego_hbm.at[src_v.at[j + 1]], rows_v.at[1 - b], sem)

                pltpu.make_async_copy(
                    ego_hbm.at[src_v.at[j]], rows_v.at[b], sem).wait()

                def _scale(g, cc):
                    wv = w_v[j, pl.ds(g * 16, 16)]
                    for l in range(16):
                        wi = wv[l]
                        i = g * 16 + l
                        for k in range(D // 16):
                            sl = pl.ds(k * 16, 16)
                            rows_v[b, i, sl] = rows_v[b, i, sl] * wi
                    return cc

                lax.fori_loop(0, C // 16, _scale, 0)

                pltpu.sync_copy(rows_v.at[b], acc_sh.at[dst_v.at[j]], add=True)
                return carry

            lax.fori_loop(0, SEG, _chunk, 0)

        plsc.subcore_barrier()
        pltpu.sync_copy(acc_sh.at[pl.ds(s * RPS, RPS)], out_hbm.at[c, s])

    return _sc_aggregate


def _tc_layer_body(p_ref, gw_ref, gb_ref, bw_ref, bb_ref, ego_ref, mlp_ref):
    side = p_ref[0] + p_ref[1]
    h = lax.dot_general(side, gw_ref[...], (((1,), (1,)), ((), ())),
                        preferred_element_type=jnp.float32) + gb_ref[...]
    h = jnp.where(h >= 0, h, 0.01 * h)
    ego_ref[...] = h
    mlp_ref[...] = lax.dot_general(h, bw_ref[...], (((1,), (1,)), ((), ())),
                                   preferred_element_type=jnp.float32) + bb_ref[...]


ROWS_BLK = 1000


def _tc_layer(p, gw, gb, bw, bb):
    return pl.pallas_call(
        _tc_layer_body,
        grid=(N // ROWS_BLK,),
        in_specs=[
            pl.BlockSpec((NC, ROWS_BLK, D), lambda i: (0, i, 0)),
            pl.BlockSpec((D, D), lambda i: (0, 0)),
            pl.BlockSpec((1, D), lambda i: (0, 0)),
            pl.BlockSpec((D, D), lambda i: (0, 0)),
            pl.BlockSpec((1, D), lambda i: (0, 0)),
        ],
        out_specs=[
            pl.BlockSpec((ROWS_BLK, D), lambda i: (i, 0)),
            pl.BlockSpec((ROWS_BLK, D), lambda i: (i, 0)),
        ],
        out_shape=[
            jax.ShapeDtypeStruct((N, D), jnp.float32),
            jax.ShapeDtypeStruct((N, D), jnp.float32),
        ],
    )(p, gw, gb, bw, bb)


def kernel(edge_index, edge_weight, emb_user, emb_item,
           gc_W0, gc_b0, gc_W1, gc_b1, bi_W0, bi_b0, bi_W1, bi_b1):
    src = edge_index[0].reshape(NW * NSEG, SEG, C)
    dst = edge_index[1].reshape(NW * NSEG, SEG, C)
    w = edge_weight.reshape(NW * NSEG, SEG, C)
    ego0 = jnp.concatenate([emb_user, emb_item], axis=0)

    sc_aggregate = _make_sc_aggregate()
    p0 = sc_aggregate(ego0, src, dst, w).reshape(NC, N, D)
    ego1, mlp0 = _tc_layer(p0, gc_W0, gc_b0.reshape(1, D), bi_W0, bi_b0.reshape(1, D))
    p1 = sc_aggregate(ego1, src, dst, w).reshape(NC, N, D)
    _, mlp1 = _tc_layer(p1, gc_W1, gc_b1.reshape(1, D), bi_W1, bi_b1.reshape(1, D))

    users = jnp.concatenate(
        [ego0[:NUM_USERS], mlp0[:NUM_USERS], mlp1[:NUM_USERS]], axis=1)
    items = jnp.concatenate(
        [ego0[NUM_USERS:], mlp0[NUM_USERS:], mlp1[NUM_USERS:]], axis=1)
    return (users, items)


# speedup vs baseline: 2.5264x; 2.5264x over previous
import functools

import jax
import jax.numpy as jnp
from jax import lax
from jax.experimental import pallas as pl
from jax.experimental.pallas import tpu as pltpu
from jax.experimental.pallas import tpu_sc as plsc

NUM_USERS = 5000
NUM_ITEMS = 5000
N = NUM_USERS + NUM_ITEMS
E = 320000
D = 128

NC = 2
NS = 16
NW = NC * NS
EPW = E // NW
C = 80
NCHUNK = EPW // C
SEG = 25
NSEG = NCHUNK // SEG
RPS = N // NS


@functools.cache
def _make_sc_aggregate():
    mesh = plsc.VectorSubcoreMesh(core_axis_name="c", subcore_axis_name="s")

    @functools.partial(
        pl.kernel,
        mesh=mesh,
        out_type=jax.ShapeDtypeStruct((NC, NS, RPS, D), jnp.float32),
        scratch_types=[
            pltpu.VMEM((SEG, C), jnp.int32),
            pltpu.VMEM((SEG, C), jnp.int32),
            pltpu.VMEM((SEG, C), jnp.float32),
            pltpu.VMEM((2, C, D), jnp.float32),
            pltpu.VMEM_SHARED((N, D), jnp.float32),
            pltpu.SemaphoreType.DMA,
        ],
    )
    def _sc_aggregate(ego_hbm, src_hbm, dst_hbm, w_hbm, out_hbm,
                      src_v, dst_v, w_v, rows_v, acc_sh, sem):
        c = lax.axis_index("c")
        s = lax.axis_index("s")
        wid = c * NS + s

        zvec = jnp.zeros((16,), jnp.float32)

        def _zrow(i, carry):
            for k in range(D // 16):
                rows_v[0, i, pl.ds(k * 16, 16)] = zvec
            return carry

        lax.fori_loop(0, C, _zrow, 0)
        base = s * RPS
        for q in range(RPS // C):
            pltpu.sync_copy(rows_v.at[0], acc_sh.at[pl.ds(base + q * C, C)])
        rem = RPS % C
        if rem:
            pltpu.sync_copy(rows_v.at[0, pl.ds(0, rem)],
                            acc_sh.at[pl.ds(base + (RPS // C) * C, rem)])
        plsc.subcore_barrier()

        for seg in range(NSEG):
            sg = wid * NSEG + seg
            cp_src = pltpu.async_copy(src_hbm.at[sg], src_v, sem)
            cp_dst = pltpu.async_copy(dst_hbm.at[sg], dst_v, sem)
            cp_w = pltpu.async_copy(w_hbm.at[sg], w_v, sem)
            cp_src.wait()
            cp_dst.wait()
            cp_w.wait()

            pltpu.async_copy(ego_hbm.at[src_v.at[0]], rows_v.at[0], sem)

            def _chunk(j, carry):
                b = lax.rem(j, 2)

                @pl.when(j + 1 < SEG)
                def _prefetch():
                    pltpu.async_copy(
                        ego_hbm.at[src_v.at[j + 1]], rows_v.at[1 - b], sem)

                pltpu.make_async_copy(
                    ego_hbm.at[pl.ds(0, C)], rows_v.at[b], sem).wait()

                def _scale_buf(bb):
                    def _scale(g, cc):
                        wv = w_v[j, pl.ds(g * 16, 16)]
                        for l in range(16):
                            wi = wv[l]
                            i = g * 16 + l
                            for k in range(D // 16):
                                sl = pl.ds(k * 16, 16)
                                rows_v[bb, i, sl] = rows_v[bb, i, sl] * wi
                        return cc

                    lax.fori_loop(0, C // 16, _scale, 0)

                @pl.when(b == 0)
                def _s0():
                    _scale_buf(0)

                @pl.when(b == 1)
                def _s1():
                    _scale_buf(1)

                pltpu.sync_copy(rows_v.at[b], acc_sh.at[dst_v.at[j]], add=True)
                return carry

            lax.fori_loop(0, SEG, _chunk, 0)

        plsc.subcore_barrier()
        pltpu.sync_copy(acc_sh.at[pl.ds(s * RPS, RPS)], out_hbm.at[c, s])

    return _sc_aggregate


def _tc_layer_body(p_ref, gw_ref, gb_ref, bw_ref, bb_ref, ego_ref, mlp_ref):
    side = p_ref[0] + p_ref[1]
    h = lax.dot_general(side, gw_ref[...], (((1,), (1,)), ((), ())),
                        preferred_element_type=jnp.float32) + gb_ref[...]
    h = jnp.where(h >= 0, h, 0.01 * h)
    ego_ref[...] = h
    mlp_ref[...] = lax.dot_general(h, bw_ref[...], (((1,), (1,)), ((), ())),
                                   preferred_element_type=jnp.float32) + bb_ref[...]


ROWS_BLK = 1000


def _tc_layer(p, gw, gb, bw, bb):
    return pl.pallas_call(
        _tc_layer_body,
        grid=(N // ROWS_BLK,),
        in_specs=[
            pl.BlockSpec((NC, ROWS_BLK, D), lambda i: (0, i, 0)),
            pl.BlockSpec((D, D), lambda i: (0, 0)),
            pl.BlockSpec((1, D), lambda i: (0, 0)),
            pl.BlockSpec((D, D), lambda i: (0, 0)),
            pl.BlockSpec((1, D), lambda i: (0, 0)),
        ],
        out_specs=[
            pl.BlockSpec((ROWS_BLK, D), lambda i: (i, 0)),
            pl.BlockSpec((ROWS_BLK, D), lambda i: (i, 0)),
        ],
        out_shape=[
            jax.ShapeDtypeStruct((N, D), jnp.float32),
            jax.ShapeDtypeStruct((N, D), jnp.float32),
        ],
    )(p, gw, gb, bw, bb)


def kernel(edge_index, edge_weight, emb_user, emb_item,
           gc_W0, gc_b0, gc_W1, gc_b1, bi_W0, bi_b0, bi_W1, bi_b1):
    src = edge_index[0].reshape(NW * NSEG, SEG, C)
    dst = edge_index[1].reshape(NW * NSEG, SEG, C)
    w = edge_weight.reshape(NW * NSEG, SEG, C)
    ego0 = jnp.concatenate([emb_user, emb_item], axis=0)

    sc_aggregate = _make_sc_aggregate()
    p0 = sc_aggregate(ego0, src, dst, w).reshape(NC, N, D)
    ego1, mlp0 = _tc_layer(p0, gc_W0, gc_b0.reshape(1, D), bi_W0, bi_b0.reshape(1, D))
    p1 = sc_aggregate(ego1, src, dst, w).reshape(NC, N, D)
    _, mlp1 = _tc_layer(p1, gc_W1, gc_b1.reshape(1, D), bi_W1, bi_b1.reshape(1, D))

    users = jnp.concatenate(
        [ego0[:NUM_USERS], mlp0[:NUM_USERS], mlp1[:NUM_USERS]], axis=1)
    items = jnp.concatenate(
        [ego0[NUM_USERS:], mlp0[NUM_USERS:], mlp1[NUM_USERS:]], axis=1)
    return (users, items)
